# baseline (speedup 1.0000x reference)
import jax
import jax.numpy as jnp
from jax.experimental import pallas as pl


def _vq_body(z_ref, w_ref, iota_ref, zq_ref, ind_ref, loss_ref):
    z = z_ref[...]
    w = w_ref[...]
    iota_f = iota_ref[...]
    w2 = jnp.sum(w * w, axis=1)[None, :]
    z2 = jnp.sum(z * z, axis=1, keepdims=True)
    wn = (w * -2.0).astype(jnp.bfloat16)
    e2 = jax.lax.dot_general(z.astype(jnp.bfloat16), wn,
                             (((1,), (1,)), ((), ())),
                             preferred_element_type=jnp.float32)
    d = (z2 + e2) + w2
    m = jnp.min(d, axis=1, keepdims=True)
    ind_f = jnp.min(jnp.where(d == m, iota_f, float(d.shape[1])),
                    axis=1, keepdims=True)
    ind_ref[0, 0, :] = ind_f[:, 0].astype(jnp.int32)
    oh = (iota_f == ind_f).astype(jnp.bfloat16)
    zq = jax.lax.dot_general(oh, w.astype(jnp.bfloat16), (((1,), (0,)), ((), ())),
                             preferred_element_type=jnp.float32)
    zq_ref[...] = z + (zq - z)
    part = jnp.broadcast_to(jnp.sum(m), (128,))

    @pl.when(pl.program_id(0) == 0)
    def _():
        loss_ref[0, 0, :] = part

    @pl.when(pl.program_id(0) != 0)
    def _():
        loss_ref[0, 0, :] += part


def _vq_pallas(zf, W, iota_f, tm, interpret=False):
    n, d_model = zf.shape
    k_cb = W.shape[0]
    g = n // tm
    out_shapes = (
        jax.ShapeDtypeStruct((n, d_model), jnp.float32),
        jax.ShapeDtypeStruct((g, 1, tm), jnp.int32),
        jax.ShapeDtypeStruct((1, 1, 128), jnp.float32),
    )
    return pl.pallas_call(
        _vq_body,
        grid=(g,),
        in_specs=[
            pl.BlockSpec((tm, d_model), lambda i: (i, 0)),
            pl.BlockSpec((k_cb, d_model), lambda i: (0, 0)),
            pl.BlockSpec((1, k_cb), lambda i: (0, 0)),
        ],
        out_specs=(
            pl.BlockSpec((tm, d_model), lambda i: (i, 0)),
            pl.BlockSpec((1, 1, tm), lambda i: (i, 0, 0)),
            pl.BlockSpec((1, 1, 128), lambda i: (0, 0, 0)),
        ),
        out_shape=out_shapes,
        interpret=interpret,
    )(zf, W, iota_f)


def kernel(z, W):
    b, t, d_model = z.shape
    n = b * t
    tm = 2304
    k_cb = W.shape[0]
    zf = z.reshape(n, d_model)
    iota_f = jax.lax.iota(jnp.float32, k_cb).reshape(1, k_cb)
    zq_st, ind3, part = _vq_pallas(zf, W, iota_f, tm)
    loss = part[0, 0, 0] * (2.0 / (n * d_model))
    return zq_st.reshape(b, t, d_model), ind3.reshape(b, t), loss
